# baseline (speedup 1.0000x reference)
import functools

import jax
import jax.numpy as jnp
from jax import lax
from jax.experimental import pallas as pl
from jax.experimental.pallas import tpu as pltpu
from jax.experimental.pallas import tpu_sc as plsc

_NC = 2
_NS = 16
_NW = _NC * _NS
_SPLITS = ((0, 96), (96, 104))
_NBUF = 4
_DP = 128


def kernel(input_ids, table):
    B, S = input_ids.shape
    V, D = table.shape
    assert B % _NW == 0
    rows_w = B // _NW
    max_sz = max(sz for _, sz in _SPLITS)

    idsf = input_ids.astype(jnp.int32).reshape(-1)
    tablep = jnp.pad(table, ((0, 0), (0, _DP - D)))
    mesh = plsc.VectorSubcoreMesh(
        core_axis_name="c", subcore_axis_name="s", num_cores=_NC, num_subcores=_NS
    )

    @functools.partial(
        pl.kernel,
        out_type=jax.ShapeDtypeStruct((B, S, D), jnp.float32),
        mesh=mesh,
        scratch_types=[
            pltpu.VMEM((rows_w * S,), jnp.int32),
            pltpu.VMEM((_NBUF, max_sz, _DP), jnp.float32),
            pltpu.SemaphoreType.DMA((_NBUF,)),
            pltpu.SemaphoreType.DMA((_NBUF,)),
        ],
        compiler_params=pltpu.CompilerParams(use_tc_tiling_on_sc=False),
    )
    def emb(ids_hbm, table_hbm, out_hbm, idx_v, rows_v, sem_g, sem_w):
        wid = lax.axis_index("s") * _NC + lax.axis_index("c")
        row0 = wid * rows_w
        pltpu.sync_copy(ids_hbm.at[pl.ds(row0 * S, rows_w * S)], idx_v)

        def gather_desc(r, h, slot):
            s0, sz = _SPLITS[h]
            return pltpu.make_async_copy(
                table_hbm.at[idx_v.at[pl.ds(r * S + s0, sz)]],
                rows_v.at[slot, pl.ds(0, sz)],
                sem_g.at[slot],
            )

        def write_desc(r, h, slot):
            s0, sz = _SPLITS[h]
            return pltpu.make_async_copy(
                rows_v.at[slot, pl.ds(0, sz), pl.ds(0, D)],
                out_hbm.at[row0 + r, pl.ds(s0, sz)],
                sem_w.at[slot],
            )

        gather_desc(0, 0, 0).start()
        gather_desc(0, 1, 1).start()

        def body(r2, carry):
            for k in range(4):
                r = 2 * r2 + k // 2
                h = k % 2
                slot_pre = (k + 2) % 4
                if k < 2:
                    @pl.when(r2 >= 1)
                    def _():
                        write_desc(2 * r2 - 1, h, slot_pre).wait()

                    gather_desc(2 * r2 + 1, h, slot_pre).start()
                else:
                    write_desc(2 * r2, h, slot_pre).wait()

                    @pl.when(r2 < rows_w // 2 - 1)
                    def _():
                        gather_desc(2 * r2 + 2, h, slot_pre).start()

                gather_desc(r, h, k).wait()
                write_desc(r, h, k).start()
            return carry

        lax.fori_loop(0, rows_w // 2, body, 0)
        write_desc(rows_w - 1, 0, 2).wait()
        write_desc(rows_w - 1, 1, 3).wait()

    return emb(idsf, tablep)


# speedup vs baseline: 1.0156x; 1.0156x over previous
import functools

import jax
import jax.numpy as jnp
from jax import lax
from jax.experimental import pallas as pl
from jax.experimental.pallas import tpu as pltpu
from jax.experimental.pallas import tpu_sc as plsc

_NC = 2
_NS = 16
_NW = _NC * _NS
_CH = 128
_NBUF = 8
_LOOK = 4


def kernel(input_ids, table):
    B, S = input_ids.shape
    V, D = table.shape
    N = B * S
    assert N % (_NW * _CH) == 0
    per_w = N // _NW
    n_chunks = per_w // _CH
    assert n_chunks % _NBUF == 0 and n_chunks >= 2 * _NBUF

    ids = input_ids.reshape(_NW, n_chunks, _CH).astype(jnp.int32)
    mesh = plsc.VectorSubcoreMesh(
        core_axis_name="c", subcore_axis_name="s", num_cores=_NC, num_subcores=_NS
    )

    @functools.partial(
        pl.kernel,
        out_type=jax.ShapeDtypeStruct((N, D), jnp.float32),
        mesh=mesh,
        scratch_types=[
            pltpu.VMEM((n_chunks, _CH), jnp.int32),
            pltpu.VMEM((_NBUF, _CH, D), jnp.float32),
            pltpu.SemaphoreType.DMA((_NBUF,)),
            pltpu.SemaphoreType.DMA((_NBUF,)),
        ],
        compiler_params=pltpu.CompilerParams(use_tc_tiling_on_sc=False),
    )
    def emb(ids_hbm, table_hbm, out_hbm, idx_v, rows_v, sem_g, sem_w):
        wid = lax.axis_index("s") * _NC + lax.axis_index("c")
        base = wid * n_chunks
        pltpu.sync_copy(ids_hbm.at[wid], idx_v)

        def gather_desc(j, slot):
            return pltpu.make_async_copy(
                table_hbm.at[idx_v.at[j]], rows_v.at[slot], sem_g.at[slot]
            )

        def write_desc(j, slot):
            return pltpu.make_async_copy(
                rows_v.at[slot],
                out_hbm.at[pl.ds((base + j) * _CH, _CH)],
                sem_w.at[slot],
            )

        for j0 in range(_LOOK):
            gather_desc(j0, j0).start()

        def body(jb, carry):
            for k in range(_NBUF):
                j = jb * _NBUF + k
                slot_pre = (k + _LOOK) % _NBUF
                if k < _LOOK:
                    @pl.when(jb >= 1)
                    def _():
                        write_desc(j - _LOOK, slot_pre).wait()

                    gather_desc(j + _LOOK, slot_pre).start()
                else:
                    write_desc(j - _LOOK, slot_pre).wait()

                    @pl.when(jb < n_chunks // _NBUF - 1)
                    def _():
                        gather_desc(j + _LOOK, slot_pre).start()

                gather_desc(j, k).wait()
                write_desc(j, k).start()
            return carry

        lax.fori_loop(0, n_chunks // _NBUF, body, 0)
        for j in range(n_chunks - _LOOK, n_chunks):
            write_desc(j, j % _NBUF).wait()

    out = emb(ids, table)
    return out.reshape(B, S, D)


# speedup vs baseline: 1.2359x; 1.2169x over previous
import functools

import jax
import jax.numpy as jnp
from jax import lax
from jax.experimental import pallas as pl
from jax.experimental.pallas import tpu as pltpu
from jax.experimental.pallas import tpu_sc as plsc

_NC = 2
_NS = 16
_NW = _NC * _NS
_CH = 128
_NBUF = 4
_DP = 128


def kernel(input_ids, table):
    B, S = input_ids.shape
    V, D = table.shape
    N = B * S
    assert N % (_NW * _CH) == 0
    per_w = N // _NW
    n_chunks = per_w // _CH

    ids = input_ids.reshape(_NW, n_chunks, _CH).astype(jnp.int32)
    tablep = jnp.pad(table, ((0, 0), (0, _DP - D)))
    mesh = plsc.VectorSubcoreMesh(
        core_axis_name="c", subcore_axis_name="s", num_cores=_NC, num_subcores=_NS
    )

    @functools.partial(
        pl.kernel,
        out_type=jax.ShapeDtypeStruct((N, _DP), jnp.float32),
        mesh=mesh,
        scratch_types=[
            pltpu.VMEM((n_chunks, _CH), jnp.int32),
            pltpu.VMEM((_NBUF, _CH, _DP), jnp.float32),
            pltpu.SemaphoreType.DMA((_NBUF,)),
            pltpu.SemaphoreType.DMA((_NBUF,)),
        ],
    )
    def emb(ids_hbm, table_hbm, out_hbm, idx_v, rows_v, sem_g, sem_w):
        wid = lax.axis_index("s") * _NC + lax.axis_index("c")
        base = wid * n_chunks
        pltpu.sync_copy(ids_hbm.at[wid], idx_v)

        def gather_desc(j, slot):
            return pltpu.make_async_copy(
                table_hbm.at[idx_v.at[j]], rows_v.at[slot], sem_g.at[slot]
            )

        def write_desc(j, slot):
            return pltpu.make_async_copy(
                rows_v.at[slot],
                out_hbm.at[pl.ds((base + j) * _CH, _CH)],
                sem_w.at[slot],
            )

        gather_desc(0, 0).start()
        gather_desc(1, 1).start()

        def body(jb, carry):
            for k in range(_NBUF):
                j = jb * _NBUF + k
                slot_pre = (k + 2) % _NBUF
                if k < 2:
                    @pl.when(jb >= 1)
                    def _():
                        write_desc(j - 2, slot_pre).wait()

                    gather_desc(j + 2, slot_pre).start()
                else:
                    write_desc(j - 2, slot_pre).wait()

                    @pl.when(jb < n_chunks // _NBUF - 1)
                    def _():
                        gather_desc(j + 2, slot_pre).start()

                gather_desc(j, k).wait()
                write_desc(j, k).start()
            return carry

        lax.fori_loop(0, n_chunks // _NBUF, body, 0)
        write_desc(n_chunks - 2, (n_chunks - 2) % _NBUF).wait()
        write_desc(n_chunks - 1, (n_chunks - 1) % _NBUF).wait()

    out = emb(ids, tablep)
    return out[:, :D].reshape(B, S, D)
